# Initial kernel scaffold; baseline (speedup 1.0000x reference)
import functools

import jax
import jax.numpy as jnp
from jax import lax
from jax.experimental import pallas as pl
from jax.experimental.pallas import tpu as pltpu
from jax.experimental.pallas import tpu_sc as plsc

N = 100000
E = 1600000
NC = 2
NS = 16
NW = NC * NS
CHUNK = 128
GROUP = 8
GSZ = GROUP * CHUNK
CT = 392
GT = CT // GROUP
EW = CT * CHUNK
EP = NW * EW
NB = 6400
NP = NS * NB
EBLK = 4096
NBLK = 3200

_mesh = plsc.VectorSubcoreMesh(core_axis_name="c", subcore_axis_name="s")


@functools.partial(
    pl.kernel,
    out_type=(jax.ShapeDtypeStruct((EP, 4), jnp.float32),
              jax.ShapeDtypeStruct((EP, 4), jnp.float32)),
    mesh=_mesh,
    scratch_types=[
        pltpu.VMEM((CT, CHUNK), jnp.int32),
        pltpu.VMEM((CT, CHUNK), jnp.int32),
        pltpu.VMEM((GSZ, 4), jnp.float32),
        pltpu.VMEM((GSZ, 4), jnp.float32),
        pltpu.SemaphoreType.DMA,
    ],
)
def _sc_gather(x_hbm, row_hbm, col_hbm, xr_hbm, xc_hbm,
               idxr, idxc, bufr, bufc, sem):
    c = lax.axis_index("c")
    s = lax.axis_index("s")
    wid = s * NC + c
    pltpu.sync_copy(row_hbm.at[wid], idxr)
    pltpu.sync_copy(col_hbm.at[wid], idxc)
    ebase = wid * EW

    def group(g, carry):
        descs = []
        for j in range(GROUP):
            ch = g * GROUP + j
            dst = pl.ds(j * CHUNK, CHUNK)
            descs.append(pltpu.async_copy(x_hbm.at[idxr.at[ch]], bufr.at[dst], sem))
            descs.append(pltpu.async_copy(x_hbm.at[idxc.at[ch]], bufc.at[dst], sem))
        for d in descs:
            d.wait()
        off = ebase + g * GSZ
        pltpu.sync_copy(bufr, xr_hbm.at[pl.ds(off, GSZ)])
        pltpu.sync_copy(bufc, xc_hbm.at[pl.ds(off, GSZ)])
        return carry

    lax.fori_loop(0, GT, group, 0)


def _make_sc_scatter(with_m: bool):
    scratch = [
        pltpu.VMEM((CT, CHUNK), jnp.int32),
        pltpu.VMEM((GSZ, 4), jnp.float32),
        pltpu.VMEM_SHARED((NP, 4), jnp.float32),
        pltpu.SemaphoreType.DMA,
    ]

    def body(*refs):
        (src_hbm, row_hbm, zeros_hbm, out_hbm,
         idxr, mbuf, acc, sem) = refs
        c = lax.axis_index("c")
        s = lax.axis_index("s")
        wid = s * NC + c
        pltpu.sync_copy(zeros_hbm, acc.at[pl.ds(s * NB, NB)])
        pltpu.sync_copy(row_hbm.at[wid], idxr)
        if not with_m:
            pltpu.sync_copy(src_hbm, mbuf)
        plsc.subcore_barrier()
        ebase = wid * EW

        def group(g, carry):
            if with_m:
                off = ebase + g * GSZ
                pltpu.sync_copy(src_hbm.at[pl.ds(off, GSZ)], mbuf)
            descs = []
            for j in range(GROUP):
                ch = g * GROUP + j
                src = mbuf.at[pl.ds(j * CHUNK, CHUNK)]
                descs.append(
                    pltpu.async_copy(src, acc.at[idxr.at[ch]], sem, add=True))
            for d in descs:
                d.wait()
            return carry

        lax.fori_loop(0, GT, group, 0)
        plsc.subcore_barrier()
        pltpu.sync_copy(acc.at[pl.ds(s * NB, NB)],
                        out_hbm.at[c, pl.ds(s * NB, NB)])

    return pl.kernel(
        body,
        out_type=jax.ShapeDtypeStruct((NC, NP, 4), jnp.float32),
        mesh=_mesh,
        scratch_types=scratch,
    )


_sc_scatter = _make_sc_scatter(True)
_sc_count = _make_sc_scatter(False)


def _mlp_body(xr_ref, xc_ref, attr_ref, w1r_ref, w1a_ref, b1_ref, w2t_ref,
              m_ref):
    d = xr_ref[...] - xc_ref[...]
    radial = jnp.sqrt(jnp.sum(d * d, axis=1, keepdims=True))
    h = (radial * w1r_ref[...]
         + jnp.dot(attr_ref[...], w1a_ref[...],
                   preferred_element_type=jnp.float32)
         + b1_ref[...])
    h = h * jax.nn.sigmoid(h)
    e = jnp.tanh(jnp.sum(h * w2t_ref[...], axis=1, keepdims=True))
    m_ref[...] = d * e


def _tc_mlp(xr, xc, attr, w1r, w1a, b1, w2t):
    grid = (EP // EBLK,)
    edge_spec = pl.BlockSpec((EBLK, 4), lambda i: (i, 0))
    w_spec = pl.BlockSpec((1, 64), lambda i: (0, 0))
    wa_spec = pl.BlockSpec((4, 64), lambda i: (0, 0))
    return pl.pallas_call(
        _mlp_body,
        grid=grid,
        in_specs=[edge_spec, edge_spec, edge_spec,
                  w_spec, wa_spec, w_spec, w_spec],
        out_specs=edge_spec,
        out_shape=jax.ShapeDtypeStruct((EP, 4), jnp.float32),
    )(xr, xc, attr, w1r, w1a, b1, w2t)


def _node_body(x_ref, p_ref, cnt_ref, v_ref, vw1_ref, vb1_ref, vw2t_ref,
               vb2_ref, o_ref):
    cnt = cnt_ref[0] + cnt_ref[1]
    agg = (p_ref[0] + p_ref[1]) / jnp.maximum(cnt, 1.0)
    vn = jnp.sqrt(jnp.sum(v_ref[...] * v_ref[...], axis=1, keepdims=True))
    hv = vn * vw1_ref[...] + vb1_ref[...]
    hv = hv * jax.nn.sigmoid(hv)
    s = jnp.sum(hv * vw2t_ref[...], axis=1, keepdims=True) + vb2_ref[0, 0]
    o_ref[...] = x_ref[...] + agg + v_ref[...] * s


def _tc_node(x, partials, cntp, vpad, vw1, vb1, vw2t, vb2):
    grid = (NP // NBLK,)
    node_spec = pl.BlockSpec((NBLK, 4), lambda i: (i, 0))
    pair_spec = pl.BlockSpec((2, NBLK, 4), lambda i: (0, i, 0))
    w_spec = pl.BlockSpec((1, 64), lambda i: (0, 0))
    s_spec = pl.BlockSpec((1, 1), lambda i: (0, 0))
    return pl.pallas_call(
        _node_body,
        grid=grid,
        in_specs=[node_spec, pair_spec, pair_spec, node_spec,
                  w_spec, w_spec, w_spec, s_spec],
        out_specs=node_spec,
        out_shape=jax.ShapeDtypeStruct((NP, 4), jnp.float32),
    )(x, partials, cntp, vpad, vw1, vb1, vw2t, vb2)


def kernel(pos, edge_index, edge_attr, v, params):
    row = edge_index[0]
    col = edge_index[1]
    rowp = jnp.concatenate(
        [row, jnp.full((EP - E,), NP - 1, jnp.int32)]).reshape(NW, CT, CHUNK)
    colp = jnp.concatenate(
        [col, jnp.zeros((EP - E,), jnp.int32)]).reshape(NW, CT, CHUNK)
    attrp = jnp.pad(edge_attr, ((0, EP - E), (0, 0)))
    x = jnp.pad(pos, ((0, NP - N), (0, 1)))
    vpad = jnp.pad(v, ((0, NP - N), (0, 1)))
    zeros_nb = jnp.zeros((NB, 4), jnp.float32)
    ones_g = jnp.ones((GSZ, 4), jnp.float32)

    cntp = _sc_count(ones_g, rowp, zeros_nb)

    for p in params:
        w1r = p['phi_W1'][0:1]
        w1a = p['phi_W1'][1:5]
        b1 = p['phi_b1'][None, :]
        w2t = p['phi_W2'].T
        vw1 = p['vel_W1']
        vb1 = p['vel_b1'][None, :]
        vw2t = p['vel_W2'].T
        vb2 = p['vel_b2'][None, :]
        xr, xc = _sc_gather(x, rowp, colp)
        m = _tc_mlp(xr, xc, attrp, w1r, w1a, b1, w2t)
        partials = _sc_scatter(m, rowp, zeros_nb)
        x = _tc_node(x, partials, cntp, vpad, vw1, vb1, vw2t, vb2)

    return x[:N, :3]


# speedup vs baseline: 3.3414x; 3.3414x over previous
import functools

import jax
import jax.numpy as jnp
from jax import lax
from jax.experimental import pallas as pl
from jax.experimental.pallas import tpu as pltpu
from jax.experimental.pallas import tpu_sc as plsc

N = 100000
E = 1600000
NC = 2
NS = 16
NW = NC * NS
W = 8
CHUNK = 128
GROUP = 8
GSZ = GROUP * CHUNK
CT = 392
GT = CT // GROUP
EW = CT * CHUNK
EP = NW * EW
NB = 6400
NP = NS * NB
EBLK = 4096
NBLK = 3200

_mesh = plsc.VectorSubcoreMesh(core_axis_name="c", subcore_axis_name="s")
_sc_params = pltpu.CompilerParams(use_tc_tiling_on_sc=False)


@functools.partial(
    pl.kernel,
    out_type=(jax.ShapeDtypeStruct((EP, W), jnp.float32),
              jax.ShapeDtypeStruct((EP, W), jnp.float32)),
    mesh=_mesh,
    scratch_types=[
        pltpu.VMEM((CT, CHUNK), jnp.int32),
        pltpu.VMEM((CT, CHUNK), jnp.int32),
        pltpu.VMEM((GSZ, W), jnp.float32),
        pltpu.VMEM((GSZ, W), jnp.float32),
        pltpu.SemaphoreType.DMA,
    ],
    compiler_params=_sc_params,
)
def _sc_gather(x_hbm, row_hbm, col_hbm, xr_hbm, xc_hbm,
               idxr, idxc, bufr, bufc, sem):
    c = lax.axis_index("c")
    s = lax.axis_index("s")
    wid = s * NC + c
    pltpu.sync_copy(row_hbm.at[wid], idxr)
    pltpu.sync_copy(col_hbm.at[wid], idxc)
    ebase = wid * EW

    def group(g, carry):
        descs = []
        for j in range(GROUP):
            ch = g * GROUP + j
            dst = pl.ds(j * CHUNK, CHUNK)
            descs.append(pltpu.async_copy(x_hbm.at[idxr.at[ch]], bufr.at[dst], sem))
            descs.append(pltpu.async_copy(x_hbm.at[idxc.at[ch]], bufc.at[dst], sem))
        for d in descs:
            d.wait()
        off = ebase + g * GSZ
        pltpu.sync_copy(bufr, xr_hbm.at[pl.ds(off, GSZ)])
        pltpu.sync_copy(bufc, xc_hbm.at[pl.ds(off, GSZ)])
        return carry

    lax.fori_loop(0, GT, group, 0)


def _make_sc_scatter(with_m: bool):
    scratch = [
        pltpu.VMEM((CT, CHUNK), jnp.int32),
        pltpu.VMEM((GSZ, W), jnp.float32),
        pltpu.VMEM_SHARED((NP, W), jnp.float32),
        pltpu.SemaphoreType.DMA,
    ]

    def body(*refs):
        (src_hbm, row_hbm, zeros_hbm, out_hbm,
         idxr, mbuf, acc, sem) = refs
        c = lax.axis_index("c")
        s = lax.axis_index("s")
        wid = s * NC + c
        pltpu.sync_copy(zeros_hbm, acc.at[pl.ds(s * NB, NB)])
        pltpu.sync_copy(row_hbm.at[wid], idxr)
        if not with_m:
            pltpu.sync_copy(src_hbm, mbuf)
        plsc.subcore_barrier()
        ebase = wid * EW

        def group(g, carry):
            if with_m:
                off = ebase + g * GSZ
                pltpu.sync_copy(src_hbm.at[pl.ds(off, GSZ)], mbuf)
            descs = []
            for j in range(GROUP):
                ch = g * GROUP + j
                src = mbuf.at[pl.ds(j * CHUNK, CHUNK)]
                descs.append(
                    pltpu.async_copy(src, acc.at[idxr.at[ch]], sem, add=True))
            for d in descs:
                d.wait()
            return carry

        lax.fori_loop(0, GT, group, 0)
        plsc.subcore_barrier()
        pltpu.sync_copy(acc.at[pl.ds(s * NB, NB)],
                        out_hbm.at[c, pl.ds(s * NB, NB)])

    return pl.kernel(
        body,
        out_type=jax.ShapeDtypeStruct((NC, NP, W), jnp.float32),
        mesh=_mesh,
        scratch_types=scratch,
        compiler_params=_sc_params,
    )


_sc_scatter = _make_sc_scatter(True)
_sc_count = _make_sc_scatter(False)


def _mlp_body(xr_ref, xc_ref, attr_ref, w1r_ref, w1a_ref, b1_ref, w2t_ref,
              m_ref):
    d = xr_ref[...] - xc_ref[...]
    radial = jnp.sqrt(jnp.sum(d * d, axis=1, keepdims=True))
    h = (radial * w1r_ref[...]
         + jnp.dot(attr_ref[...], w1a_ref[...],
                   preferred_element_type=jnp.float32)
         + b1_ref[...])
    h = h * jax.nn.sigmoid(h)
    e = jnp.tanh(jnp.sum(h * w2t_ref[...], axis=1, keepdims=True))
    m_ref[...] = d * e


def _tc_mlp(xr, xc, attr, w1r, w1a, b1, w2t):
    grid = (EP // EBLK,)
    edge_spec = pl.BlockSpec((EBLK, W), lambda i: (i, 0))
    attr_spec = pl.BlockSpec((EBLK, 4), lambda i: (i, 0))
    w_spec = pl.BlockSpec((1, 64), lambda i: (0, 0))
    wa_spec = pl.BlockSpec((4, 64), lambda i: (0, 0))
    return pl.pallas_call(
        _mlp_body,
        grid=grid,
        in_specs=[edge_spec, edge_spec, attr_spec,
                  w_spec, wa_spec, w_spec, w_spec],
        out_specs=edge_spec,
        out_shape=jax.ShapeDtypeStruct((EP, W), jnp.float32),
    )(xr, xc, attr, w1r, w1a, b1, w2t)


def _node_body(x_ref, p_ref, cnt_ref, v_ref, vw1_ref, vb1_ref, vw2t_ref,
               vb2_ref, o_ref):
    cnt = cnt_ref[0] + cnt_ref[1]
    agg = (p_ref[0] + p_ref[1]) / jnp.maximum(cnt, 1.0)
    vn = jnp.sqrt(jnp.sum(v_ref[...] * v_ref[...], axis=1, keepdims=True))
    hv = vn * vw1_ref[...] + vb1_ref[...]
    hv = hv * jax.nn.sigmoid(hv)
    s = jnp.sum(hv * vw2t_ref[...], axis=1, keepdims=True) + vb2_ref[0, 0]
    o_ref[...] = x_ref[...] + agg + v_ref[...] * s


def _tc_node(x, partials, cntp, vpad, vw1, vb1, vw2t, vb2):
    grid = (NP // NBLK,)
    node_spec = pl.BlockSpec((NBLK, W), lambda i: (i, 0))
    v_spec = node_spec
    pair_spec = pl.BlockSpec((2, NBLK, W), lambda i: (0, i, 0))
    w_spec = pl.BlockSpec((1, 64), lambda i: (0, 0))
    s_spec = pl.BlockSpec((1, 1), lambda i: (0, 0))
    return pl.pallas_call(
        _node_body,
        grid=grid,
        in_specs=[node_spec, pair_spec, pair_spec, v_spec,
                  w_spec, w_spec, w_spec, s_spec],
        out_specs=node_spec,
        out_shape=jax.ShapeDtypeStruct((NP, W), jnp.float32),
    )(x, partials, cntp, vpad, vw1, vb1, vw2t, vb2)


def kernel(pos, edge_index, edge_attr, v, params):
    row = edge_index[0]
    col = edge_index[1]
    rowp = jnp.concatenate(
        [row, jnp.full((EP - E,), NP - 1, jnp.int32)]).reshape(NW, CT, CHUNK)
    colp = jnp.concatenate(
        [col, jnp.zeros((EP - E,), jnp.int32)]).reshape(NW, CT, CHUNK)
    attrp = jnp.pad(edge_attr, ((0, EP - E), (0, 0)))
    x = jnp.pad(pos, ((0, NP - N), (0, W - 3)))
    vpad = jnp.pad(v, ((0, NP - N), (0, W - 3)))
    zeros_nb = jnp.zeros((NB, W), jnp.float32)
    ones_g = jnp.ones((GSZ, W), jnp.float32)

    cntp = _sc_count(ones_g, rowp, zeros_nb)

    for p in params:
        w1r = p['phi_W1'][0:1]
        w1a = p['phi_W1'][1:5]
        b1 = p['phi_b1'][None, :]
        w2t = p['phi_W2'].T
        vw1 = p['vel_W1']
        vb1 = p['vel_b1'][None, :]
        vw2t = p['vel_W2'].T
        vb2 = p['vel_b2'][None, :]
        xr, xc = _sc_gather(x, rowp, colp)
        m = _tc_mlp(xr, xc, attrp, w1r, w1a, b1, w2t)
        partials = _sc_scatter(m, rowp, zeros_nb)
        x = _tc_node(x, partials, cntp, vpad, vw1, vb1, vw2t, vb2)

    return x[:N, :3]


# speedup vs baseline: 3.8245x; 1.1446x over previous
import functools

import jax
import jax.numpy as jnp
from jax import lax
from jax.experimental import pallas as pl
from jax.experimental.pallas import tpu as pltpu
from jax.experimental.pallas import tpu_sc as plsc

N = 100000
E = 1600000
NC = 2
NS = 16
NW = NC * NS
W = 8
CHUNK = 128
GROUP = 8
GSZ = GROUP * CHUNK
CT = 392
GT = CT // GROUP
EW = CT * CHUNK
EP = NW * EW
NB = 6400
NP = NS * NB
EBLK = 6400
NBLK = 3200

_mesh = plsc.VectorSubcoreMesh(core_axis_name="c", subcore_axis_name="s")
_sc_params = pltpu.CompilerParams(use_tc_tiling_on_sc=False)


@functools.partial(
    pl.kernel,
    out_type=(jax.ShapeDtypeStruct((EP, W), jnp.float32),
              jax.ShapeDtypeStruct((EP, W), jnp.float32)),
    mesh=_mesh,
    scratch_types=[
        pltpu.VMEM((CT, CHUNK), jnp.int32),
        pltpu.VMEM((CT, CHUNK), jnp.int32),
        pltpu.VMEM((GSZ, W), jnp.float32),
        pltpu.VMEM((GSZ, W), jnp.float32),
        pltpu.SemaphoreType.DMA,
    ],
    compiler_params=_sc_params,
)
def _sc_gather(x_hbm, row_hbm, col_hbm, xr_hbm, xc_hbm,
               idxr, idxc, bufr, bufc, sem):
    c = lax.axis_index("c")
    s = lax.axis_index("s")
    wid = s * NC + c
    pltpu.sync_copy(row_hbm.at[wid], idxr)
    pltpu.sync_copy(col_hbm.at[wid], idxc)
    ebase = wid * EW

    def group(g, carry):
        descs = []
        for j in range(GROUP):
            ch = g * GROUP + j
            dst = pl.ds(j * CHUNK, CHUNK)
            descs.append(pltpu.async_copy(x_hbm.at[idxr.at[ch]], bufr.at[dst], sem))
            descs.append(pltpu.async_copy(x_hbm.at[idxc.at[ch]], bufc.at[dst], sem))
        for d in descs:
            d.wait()
        off = ebase + g * GSZ
        pltpu.sync_copy(bufr, xr_hbm.at[pl.ds(off, GSZ)])
        pltpu.sync_copy(bufc, xc_hbm.at[pl.ds(off, GSZ)])
        return carry

    lax.fori_loop(0, GT, group, 0)


def _make_sc_scatter(with_m: bool):
    scratch = [
        pltpu.VMEM((CT, CHUNK), jnp.int32),
        pltpu.VMEM((GSZ, W), jnp.float32),
        pltpu.VMEM_SHARED((NP, W), jnp.float32),
        pltpu.SemaphoreType.DMA,
    ]

    def body(*refs):
        (src_hbm, row_hbm, zeros_hbm, out_hbm,
         idxr, mbuf, acc, sem) = refs
        c = lax.axis_index("c")
        s = lax.axis_index("s")
        wid = s * NC + c
        pltpu.sync_copy(zeros_hbm, acc.at[pl.ds(s * NB, NB)])
        pltpu.sync_copy(row_hbm.at[wid], idxr)
        if not with_m:
            pltpu.sync_copy(src_hbm, mbuf)
        plsc.subcore_barrier()
        ebase = wid * EW

        def group(g, carry):
            if with_m:
                off = ebase + g * GSZ
                pltpu.sync_copy(src_hbm.at[pl.ds(off, GSZ)], mbuf)
            descs = []
            for j in range(GROUP):
                ch = g * GROUP + j
                src = mbuf.at[pl.ds(j * CHUNK, CHUNK)]
                descs.append(
                    pltpu.async_copy(src, acc.at[idxr.at[ch]], sem, add=True))
            for d in descs:
                d.wait()
            return carry

        lax.fori_loop(0, GT, group, 0)
        plsc.subcore_barrier()
        pltpu.sync_copy(acc.at[pl.ds(s * NB, NB)],
                        out_hbm.at[c, pl.ds(s * NB, NB)])

    return pl.kernel(
        body,
        out_type=jax.ShapeDtypeStruct((NC, NP, W), jnp.float32),
        mesh=_mesh,
        scratch_types=scratch,
        compiler_params=_sc_params,
    )


_sc_scatter = _make_sc_scatter(True)
_sc_count = _make_sc_scatter(False)


PBLK = 6400


def _prep_body(pT_ref, vT_ref, x_ref, v_ref):
    pt = jnp.transpose(pT_ref[...])
    vt = jnp.transpose(vT_ref[...])
    z = jnp.zeros((PBLK, W - 3), jnp.float32)
    x_ref[...] = jnp.concatenate([pt, z], axis=1)
    v_ref[...] = jnp.concatenate([vt, z], axis=1)


def _tc_prep(posT, vT):
    grid = (16,)
    in_spec = pl.BlockSpec((3, PBLK), lambda i: (0, i))
    out_spec = pl.BlockSpec((PBLK, W), lambda i: (i, 0))
    return pl.pallas_call(
        _prep_body,
        grid=grid,
        in_specs=[in_spec, in_spec],
        out_specs=(out_spec, out_spec),
        out_shape=(jax.ShapeDtypeStruct((NP, W), jnp.float32),
                   jax.ShapeDtypeStruct((NP, W), jnp.float32)),
    )(posT, vT)


def _mlp_body(xr_ref, xc_ref, attr_ref, w1r_ref, w1a_ref, b1_ref, w2t_ref,
              m_ref):
    d = xr_ref[...] - xc_ref[...]
    radial = jnp.sqrt(jnp.sum(d * d, axis=1, keepdims=True))
    h = (radial * w1r_ref[...]
         + jnp.dot(attr_ref[...], w1a_ref[...],
                   preferred_element_type=jnp.float32)
         + b1_ref[...])
    h = h * jax.nn.sigmoid(h)
    e = jnp.tanh(jnp.sum(h * w2t_ref[...], axis=1, keepdims=True))
    m_ref[...] = d * e


def _tc_mlp(xr, xc, attr, w1r, w1a, b1, w2t):
    grid = (E // EBLK,)
    edge_spec = pl.BlockSpec((EBLK, W), lambda i: (i, 0))
    attr_spec = pl.BlockSpec((EBLK, 4), lambda i: (i, 0))
    w_spec = pl.BlockSpec((1, 64), lambda i: (0, 0))
    wa_spec = pl.BlockSpec((4, 64), lambda i: (0, 0))
    return pl.pallas_call(
        _mlp_body,
        grid=grid,
        in_specs=[edge_spec, edge_spec, attr_spec,
                  w_spec, wa_spec, w_spec, w_spec],
        out_specs=edge_spec,
        out_shape=jax.ShapeDtypeStruct((EP, W), jnp.float32),
    )(xr, xc, attr, w1r, w1a, b1, w2t)


def _node_body(x_ref, p_ref, cnt_ref, v_ref, vw1_ref, vb1_ref, vw2t_ref,
               vb2_ref, o_ref):
    cnt = cnt_ref[0] + cnt_ref[1]
    agg = (p_ref[0] + p_ref[1]) / jnp.maximum(cnt, 1.0)
    vn = jnp.sqrt(jnp.sum(v_ref[...] * v_ref[...], axis=1, keepdims=True))
    hv = vn * vw1_ref[...] + vb1_ref[...]
    hv = hv * jax.nn.sigmoid(hv)
    s = jnp.sum(hv * vw2t_ref[...], axis=1, keepdims=True) + vb2_ref[0, 0]
    o_ref[...] = x_ref[...] + agg + v_ref[...] * s


def _tc_node(x, partials, cntp, vpad, vw1, vb1, vw2t, vb2):
    grid = (NP // NBLK,)
    node_spec = pl.BlockSpec((NBLK, W), lambda i: (i, 0))
    v_spec = node_spec
    pair_spec = pl.BlockSpec((2, NBLK, W), lambda i: (0, i, 0))
    w_spec = pl.BlockSpec((1, 64), lambda i: (0, 0))
    s_spec = pl.BlockSpec((1, 1), lambda i: (0, 0))
    return pl.pallas_call(
        _node_body,
        grid=grid,
        in_specs=[node_spec, pair_spec, pair_spec, v_spec,
                  w_spec, w_spec, w_spec, s_spec],
        out_specs=node_spec,
        out_shape=jax.ShapeDtypeStruct((NP, W), jnp.float32),
    )(x, partials, cntp, vpad, vw1, vb1, vw2t, vb2)


def kernel(pos, edge_index, edge_attr, v, params):
    row = edge_index[0]
    col = edge_index[1]
    rowp = jnp.concatenate(
        [row, jnp.full((EP - E,), NP - 1, jnp.int32)]).reshape(NW, CT, CHUNK)
    colp = jnp.concatenate(
        [col, jnp.zeros((EP - E,), jnp.int32)]).reshape(NW, CT, CHUNK)
    x, vpad = _tc_prep(jnp.pad(pos.T, ((0, 0), (0, NP - N))),
                       jnp.pad(v.T, ((0, 0), (0, NP - N))))
    zeros_nb = jnp.zeros((NB, W), jnp.float32)
    ones_g = jnp.ones((GSZ, W), jnp.float32)

    cntp = _sc_count(ones_g, rowp, zeros_nb)

    for p in params:
        w1r = p['phi_W1'][0:1]
        w1a = p['phi_W1'][1:5]
        b1 = p['phi_b1'][None, :]
        w2t = p['phi_W2'].T
        vw1 = p['vel_W1']
        vb1 = p['vel_b1'][None, :]
        vw2t = p['vel_W2'].T
        vb2 = p['vel_b2'][None, :]
        xr, xc = _sc_gather(x, rowp, colp)
        m = _tc_mlp(xr, xc, edge_attr, w1r, w1a, b1, w2t)
        partials = _sc_scatter(m, rowp, zeros_nb)
        x = _tc_node(x, partials, cntp, vpad, vw1, vb1, vw2t, vb2)

    return x[:N, :3]


# speedup vs baseline: 9.7451x; 2.5481x over previous
import functools

import jax
import jax.numpy as jnp
from jax import lax
from jax.experimental import pallas as pl
from jax.experimental.pallas import tpu as pltpu
from jax.experimental.pallas import tpu_sc as plsc

N = 100000
E = 1600000
NC = 2
NS = 16
NW = NC * NS
W = 8
CHUNK = 128
GROUP = 8
GSZ = GROUP * CHUNK
CT = 392
GT = CT // GROUP
EW = CT * CHUNK
EP = NW * EW
EPR = EP * W // 128
NB = 6400
NP = NS * NB
NPR = NP * W // 128
EBLK = 6400
EBR = EBLK * W // 128
NBLK = 3200
NBR = NBLK * W // 128
PBLK = 6400

_mesh = plsc.VectorSubcoreMesh(core_axis_name="c", subcore_axis_name="s")
_sc_params = pltpu.CompilerParams(use_tc_tiling_on_sc=False)


@functools.partial(
    pl.kernel,
    out_type=(jax.ShapeDtypeStruct((EP, W), jnp.float32),
              jax.ShapeDtypeStruct((EP, W), jnp.float32)),
    mesh=_mesh,
    scratch_types=[
        pltpu.VMEM((CT, CHUNK), jnp.int32),
        pltpu.VMEM((CT, CHUNK), jnp.int32),
        pltpu.VMEM((GSZ, W), jnp.float32),
        pltpu.VMEM((GSZ, W), jnp.float32),
        pltpu.SemaphoreType.DMA,
    ],
    compiler_params=_sc_params,
)
def _sc_gather(x_hbm, row_hbm, col_hbm, xr_hbm, xc_hbm,
               idxr, idxc, bufr, bufc, sem):
    c = lax.axis_index("c")
    s = lax.axis_index("s")
    wid = s * NC + c
    pltpu.sync_copy(row_hbm.at[wid], idxr)
    pltpu.sync_copy(col_hbm.at[wid], idxc)
    ebase = wid * EW

    def group(g, carry):
        descs = []
        for j in range(GROUP):
            ch = g * GROUP + j
            dst = pl.ds(j * CHUNK, CHUNK)
            descs.append(pltpu.async_copy(x_hbm.at[idxr.at[ch]], bufr.at[dst], sem))
            descs.append(pltpu.async_copy(x_hbm.at[idxc.at[ch]], bufc.at[dst], sem))
        for d in descs:
            d.wait()
        off = ebase + g * GSZ
        pltpu.sync_copy(bufr, xr_hbm.at[pl.ds(off, GSZ)])
        pltpu.sync_copy(bufc, xc_hbm.at[pl.ds(off, GSZ)])
        return carry

    lax.fori_loop(0, GT, group, 0)


def _make_sc_scatter(with_m: bool):
    scratch = [
        pltpu.VMEM((CT, CHUNK), jnp.int32),
        pltpu.VMEM((GSZ, W), jnp.float32),
        pltpu.VMEM_SHARED((NP, W), jnp.float32),
        pltpu.SemaphoreType.DMA,
    ]

    def body(*refs):
        (src_hbm, row_hbm, zeros_hbm, out_hbm,
         idxr, mbuf, acc, sem) = refs
        c = lax.axis_index("c")
        s = lax.axis_index("s")
        wid = s * NC + c
        pltpu.sync_copy(zeros_hbm, acc.at[pl.ds(s * NB, NB)])
        pltpu.sync_copy(row_hbm.at[wid], idxr)
        if not with_m:
            pltpu.sync_copy(src_hbm, mbuf)
        plsc.subcore_barrier()
        ebase = wid * EW

        def group(g, carry):
            if with_m:
                off = ebase + g * GSZ
                pltpu.sync_copy(src_hbm.at[pl.ds(off, GSZ)], mbuf)
            descs = []
            for j in range(GROUP):
                ch = g * GROUP + j
                src = mbuf.at[pl.ds(j * CHUNK, CHUNK)]
                descs.append(
                    pltpu.async_copy(src, acc.at[idxr.at[ch]], sem, add=True))
            for d in descs:
                d.wait()
            return carry

        lax.fori_loop(0, GT, group, 0)
        plsc.subcore_barrier()
        pltpu.sync_copy(acc.at[pl.ds(s * NB, NB)],
                        out_hbm.at[c, pl.ds(s * NB, NB)])

    return pl.kernel(
        body,
        out_type=jax.ShapeDtypeStruct((NC, NP, W), jnp.float32),
        mesh=_mesh,
        scratch_types=scratch,
        compiler_params=_sc_params,
    )


_sc_scatter = _make_sc_scatter(True)
_sc_count = _make_sc_scatter(False)


def _prep_body(pT_ref, vT_ref, x_ref, v_ref):
    pt = jnp.transpose(pT_ref[...])
    vt = jnp.transpose(vT_ref[...])
    z = jnp.zeros((PBLK, W - 3), jnp.float32)
    x_ref[...] = jnp.concatenate([pt, z], axis=1)
    v_ref[...] = jnp.concatenate([vt, z], axis=1)


def _tc_prep(posT, vT):
    grid = (NP // PBLK,)
    in_spec = pl.BlockSpec((3, PBLK), lambda i: (0, i))
    out_spec = pl.BlockSpec((PBLK, W), lambda i: (i, 0))
    return pl.pallas_call(
        _prep_body,
        grid=grid,
        in_specs=[in_spec, in_spec],
        out_specs=(out_spec, out_spec),
        out_shape=(jax.ShapeDtypeStruct((NP, W), jnp.float32),
                   jax.ShapeDtypeStruct((NP, W), jnp.float32)),
    )(posT, vT)


def _slot0_mask(rows):
    lane = jax.lax.broadcasted_iota(jnp.int32, (rows, 128), 1)
    return jnp.where(lane % W == 0, 1.0, 0.0).astype(jnp.float32)


def _mlp_body(xr_ref, xc_ref, attr_ref, sum8_ref, w1k_ref, b1k_ref, w2k_ref,
              m_ref):
    d = xr_ref[...] - xc_ref[...]
    r2 = jnp.dot(d * d, sum8_ref[...], preferred_element_type=jnp.float32)
    radial = jnp.sqrt(r2)
    e_in = radial * _slot0_mask(EBR) + attr_ref[...]
    h = jnp.dot(e_in, w1k_ref[...], preferred_element_type=jnp.float32)
    h = h + b1k_ref[...]
    h = h * jax.nn.sigmoid(h)
    s = jnp.dot(h, w2k_ref[...], preferred_element_type=jnp.float32)
    e = jnp.tanh(s)
    m_ref[...] = d * e


def _tc_mlp(xr, xc, attr_i, sum8, w1k, b1k, w2k):
    grid = (E // EBLK,)
    edge_spec = pl.BlockSpec((EBR, 128), lambda i: (i, 0))
    return pl.pallas_call(
        _mlp_body,
        grid=grid,
        in_specs=[edge_spec, edge_spec, edge_spec,
                  pl.BlockSpec((128, 128), lambda i: (0, 0)),
                  pl.BlockSpec((128, 1024), lambda i: (0, 0)),
                  pl.BlockSpec((1, 1024), lambda i: (0, 0)),
                  pl.BlockSpec((1024, 128), lambda i: (0, 0))],
        out_specs=edge_spec,
        out_shape=jax.ShapeDtypeStruct((EPR, 128), jnp.float32),
    )(xr, xc, attr_i, sum8, w1k, b1k, w2k)


def _node_body(x_ref, p0_ref, p1_ref, c0_ref, c1_ref, v_ref, sum8_ref,
               vw1k_ref, vb1k_ref, vw2k_ref, vb2_ref, o_ref):
    cnt = c0_ref[...] + c1_ref[...]
    agg = (p0_ref[...] + p1_ref[...]) / jnp.maximum(cnt, 1.0)
    v = v_ref[...]
    vn = jnp.sqrt(jnp.dot(v * v, sum8_ref[...],
                          preferred_element_type=jnp.float32))
    e_in = vn * _slot0_mask(NBR)
    hv = jnp.dot(e_in, vw1k_ref[...], preferred_element_type=jnp.float32)
    hv = hv + vb1k_ref[...]
    hv = hv * jax.nn.sigmoid(hv)
    s = jnp.dot(hv, vw2k_ref[...], preferred_element_type=jnp.float32)
    o_ref[...] = x_ref[...] + agg + v * (s + vb2_ref[0, 0])


def _tc_node(x, partials, cntp, vpad, sum8, vw1k, vb1k, vw2k, vb2):
    grid = (NP // NBLK,)
    node_spec = pl.BlockSpec((NBR, 128), lambda i: (i, 0))
    hi_spec = pl.BlockSpec((NBR, 128), lambda i: (i + NPR // NBR, 0))
    return pl.pallas_call(
        _node_body,
        grid=grid,
        in_specs=[node_spec, node_spec, hi_spec, node_spec, hi_spec,
                  node_spec,
                  pl.BlockSpec((128, 128), lambda i: (0, 0)),
                  pl.BlockSpec((128, 1024), lambda i: (0, 0)),
                  pl.BlockSpec((1, 1024), lambda i: (0, 0)),
                  pl.BlockSpec((1024, 128), lambda i: (0, 0)),
                  pl.BlockSpec((1, 1), lambda i: (0, 0))],
        out_specs=node_spec,
        out_shape=jax.ShapeDtypeStruct((NPR, 128), jnp.float32),
    )(x, partials, partials, cntp, cntp, vpad, sum8, vw1k, vb1k, vw2k, vb2)


def kernel(pos, edge_index, edge_attr, v, params):
    row = edge_index[0]
    col = edge_index[1]
    rowp = jnp.concatenate(
        [row, jnp.full((EP - E,), NP - 1, jnp.int32)]).reshape(NW, CT, CHUNK)
    colp = jnp.concatenate(
        [col, jnp.zeros((EP - E,), jnp.int32)]).reshape(NW, CT, CHUNK)
    x8, v8 = _tc_prep(jnp.pad(pos.T, ((0, 0), (0, NP - N))),
                      jnp.pad(v.T, ((0, 0), (0, NP - N))))
    x = jnp.reshape(x8, (NPR, 128))
    vpad = jnp.reshape(v8, (NPR, 128))
    attr_i = jnp.reshape(jnp.pad(edge_attr, ((0, EP - E), (1, W - 5))),
                         (EPR, 128))
    zeros_nb = jnp.zeros((NB, W), jnp.float32)
    ones_g = jnp.ones((GSZ, W), jnp.float32)
    eye16 = jnp.eye(16, dtype=jnp.float32)
    sum8 = jnp.kron(eye16, jnp.ones((W, W), jnp.float32))

    cntp8 = _sc_count(ones_g, rowp, zeros_nb)
    cntp = jnp.reshape(cntp8, (NC * NPR, 128))

    for p in params:
        w1k = jnp.kron(eye16, jnp.pad(p['phi_W1'], ((0, W - 5), (0, 0))))
        b1k = jnp.tile(p['phi_b1'], 16)[None, :]
        w2k = jnp.kron(eye16,
                       jnp.broadcast_to(p['phi_W2'], (64, W)))
        vw1k = jnp.kron(eye16, jnp.pad(p['vel_W1'], ((0, W - 1), (0, 0))))
        vb1k = jnp.tile(p['vel_b1'], 16)[None, :]
        vw2k = jnp.kron(eye16,
                        jnp.broadcast_to(p['vel_W2'], (64, W)))
        vb2 = p['vel_b2'][None, :]
        xr8, xc8 = _sc_gather(jnp.reshape(x, (NP, W)), rowp, colp)
        m = _tc_mlp(jnp.reshape(xr8, (EPR, 128)), jnp.reshape(xc8, (EPR, 128)),
                    attr_i, sum8, w1k, b1k, w2k)
        partials8 = _sc_scatter(jnp.reshape(m, (EP, W)), rowp, zeros_nb)
        partials = jnp.reshape(partials8, (NC * NPR, 128))
        x = _tc_node(x, partials, cntp, vpad, sum8, vw1k, vb1k, vw2k, vb2)

    return jnp.reshape(x, (NP, W))[:N, :3]


# speedup vs baseline: 12.4613x; 1.2787x over previous
import functools

import jax
import jax.numpy as jnp
from jax import lax
from jax.experimental import pallas as pl
from jax.experimental.pallas import tpu as pltpu
from jax.experimental.pallas import tpu_sc as plsc

N = 100000
E = 1600000
NC = 2
NS = 16
NW = NC * NS
W = 8
CHUNK = 128
GROUP = 8
GSZ = GROUP * CHUNK
CT = 392
GT = CT // GROUP
EW = CT * CHUNK
EP = NW * EW
EPR = EP * W // 128
NB = 6400
NP = NS * NB
NPR = NP * W // 128
EBLK = 6400
EBR = EBLK * W // 128
NBLK = 3200
NBR = NBLK * W // 128
PBLK = 6400

_mesh = plsc.VectorSubcoreMesh(core_axis_name="c", subcore_axis_name="s")
_sc_params = pltpu.CompilerParams(use_tc_tiling_on_sc=False)


@functools.partial(
    pl.kernel,
    out_type=(jax.ShapeDtypeStruct((EP, W), jnp.float32),
              jax.ShapeDtypeStruct((EP, W), jnp.float32)),
    mesh=_mesh,
    scratch_types=[
        pltpu.VMEM((CT, CHUNK), jnp.int32),
        pltpu.VMEM((CT, CHUNK), jnp.int32),
        pltpu.VMEM((GSZ, W), jnp.float32),
        pltpu.VMEM((GSZ, W), jnp.float32),
        pltpu.SemaphoreType.DMA,
    ],
    compiler_params=_sc_params,
)
def _sc_gather(x_hbm, row_hbm, col_hbm, xr_hbm, xc_hbm,
               idxr, idxc, bufr, bufc, sem):
    c = lax.axis_index("c")
    s = lax.axis_index("s")
    wid = s * NC + c
    pltpu.sync_copy(row_hbm.at[wid], idxr)
    pltpu.sync_copy(col_hbm.at[wid], idxc)
    ebase = wid * EW

    def group(g, carry):
        descs = []
        for j in range(GROUP):
            ch = g * GROUP + j
            dst = pl.ds(j * CHUNK, CHUNK)
            descs.append(pltpu.async_copy(x_hbm.at[idxr.at[ch]], bufr.at[dst], sem))
            descs.append(pltpu.async_copy(x_hbm.at[idxc.at[ch]], bufc.at[dst], sem))
        for d in descs:
            d.wait()
        off = ebase + g * GSZ
        pltpu.sync_copy(bufr, xr_hbm.at[pl.ds(off, GSZ)])
        pltpu.sync_copy(bufc, xc_hbm.at[pl.ds(off, GSZ)])
        return carry

    lax.fori_loop(0, GT, group, 0)


def _make_sc_scatter(with_m: bool):
    scratch = [
        pltpu.VMEM((CT, CHUNK), jnp.int32),
        pltpu.VMEM((GSZ, W), jnp.float32),
        pltpu.VMEM_SHARED((NP, W), jnp.float32),
        pltpu.SemaphoreType.DMA,
    ]

    def body(*refs):
        (src_hbm, row_hbm, zeros_hbm, out_hbm,
         idxr, mbuf, acc, sem) = refs
        c = lax.axis_index("c")
        s = lax.axis_index("s")
        wid = s * NC + c
        pltpu.sync_copy(zeros_hbm, acc.at[pl.ds(s * NB, NB)])
        pltpu.sync_copy(row_hbm.at[wid], idxr)
        if not with_m:
            pltpu.sync_copy(src_hbm, mbuf)
        plsc.subcore_barrier()
        ebase = wid * EW

        def group(g, carry):
            if with_m:
                off = ebase + g * GSZ
                pltpu.sync_copy(src_hbm.at[pl.ds(off, GSZ)], mbuf)
            descs = []
            for j in range(GROUP):
                ch = g * GROUP + j
                src = mbuf.at[pl.ds(j * CHUNK, CHUNK)]
                descs.append(
                    pltpu.async_copy(src, acc.at[idxr.at[ch]], sem, add=True))
            for d in descs:
                d.wait()
            return carry

        lax.fori_loop(0, GT, group, 0)
        plsc.subcore_barrier()
        pltpu.sync_copy(acc.at[pl.ds(s * NB, NB)],
                        out_hbm.at[c, pl.ds(s * NB, NB)])

    return pl.kernel(
        body,
        out_type=jax.ShapeDtypeStruct((NC, NP, W), jnp.float32),
        mesh=_mesh,
        scratch_types=scratch,
        compiler_params=_sc_params,
    )


_sc_scatter = _make_sc_scatter(True)
_sc_count = _make_sc_scatter(False)


def _prep_body(pT_ref, vT_ref, x_ref, v_ref):
    pt = jnp.transpose(pT_ref[...])
    vt = jnp.transpose(vT_ref[...])
    z = jnp.zeros((PBLK, W - 3), jnp.float32)
    x_ref[...] = jnp.concatenate([pt, z], axis=1)
    v_ref[...] = jnp.concatenate([vt, z], axis=1)


def _tc_prep(posT, vT):
    grid = (NP // PBLK,)
    in_spec = pl.BlockSpec((3, PBLK), lambda i: (0, i))
    out_spec = pl.BlockSpec((PBLK, W), lambda i: (i, 0))
    return pl.pallas_call(
        _prep_body,
        grid=grid,
        in_specs=[in_spec, in_spec],
        out_specs=(out_spec, out_spec),
        out_shape=(jax.ShapeDtypeStruct((NP, W), jnp.float32),
                   jax.ShapeDtypeStruct((NP, W), jnp.float32)),
    )(posT, vT)


def _slot0_mask(rows):
    lane = jax.lax.broadcasted_iota(jnp.int32, (rows, 128), 1)
    return jnp.where(lane % W == 0, 1.0, 0.0).astype(jnp.float32)


def _mlp_body(xr_ref, xc_ref, attr_ref, sum8_ref, w1k_ref, b1k_ref, w2k_ref,
              m_ref):
    d = xr_ref[...] - xc_ref[...]
    r2 = jnp.dot(d * d, sum8_ref[...], preferred_element_type=jnp.float32)
    radial = jnp.sqrt(r2)
    e_in = radial * _slot0_mask(EBR) + attr_ref[...].astype(jnp.float32)
    h = jnp.dot(e_in, w1k_ref[...], preferred_element_type=jnp.float32)
    h = h + b1k_ref[...]
    h = h * jax.nn.sigmoid(h)
    s = jnp.dot(h, w2k_ref[...], preferred_element_type=jnp.float32)
    e = jnp.tanh(s)
    m_ref[...] = d * e


def _tc_mlp(xr, xc, attr_i, sum8, w1k, b1k, w2k):
    grid = (E // EBLK,)
    edge_spec = pl.BlockSpec((EBR, 128), lambda i: (i, 0))
    return pl.pallas_call(
        _mlp_body,
        grid=grid,
        in_specs=[edge_spec, edge_spec, edge_spec,
                  pl.BlockSpec((128, 128), lambda i: (0, 0)),
                  pl.BlockSpec((128, 1024), lambda i: (0, 0)),
                  pl.BlockSpec((1, 1024), lambda i: (0, 0)),
                  pl.BlockSpec((1024, 128), lambda i: (0, 0))],
        out_specs=edge_spec,
        out_shape=jax.ShapeDtypeStruct((EPR, 128), jnp.float32),
    )(xr, xc, attr_i, sum8, w1k, b1k, w2k)


def _node_body(x_ref, p0_ref, p1_ref, c0_ref, c1_ref, v_ref, sum8_ref,
               vw1k_ref, vb1k_ref, vw2k_ref, vb2_ref, o_ref):
    cnt = c0_ref[...] + c1_ref[...]
    agg = (p0_ref[...] + p1_ref[...]) / jnp.maximum(cnt, 1.0)
    v = v_ref[...]
    vn = jnp.sqrt(jnp.dot(v * v, sum8_ref[...],
                          preferred_element_type=jnp.float32))
    e_in = vn * _slot0_mask(NBR)
    hv = jnp.dot(e_in, vw1k_ref[...], preferred_element_type=jnp.float32)
    hv = hv + vb1k_ref[...]
    hv = hv * jax.nn.sigmoid(hv)
    s = jnp.dot(hv, vw2k_ref[...], preferred_element_type=jnp.float32)
    o_ref[...] = x_ref[...] + agg + v * (s + vb2_ref[0, 0])


def _tc_node(x, partials, cntp, vpad, sum8, vw1k, vb1k, vw2k, vb2):
    grid = (NP // NBLK,)
    node_spec = pl.BlockSpec((NBR, 128), lambda i: (i, 0))
    hi_spec = pl.BlockSpec((NBR, 128), lambda i: (i + NPR // NBR, 0))
    return pl.pallas_call(
        _node_body,
        grid=grid,
        in_specs=[node_spec, node_spec, hi_spec, node_spec, hi_spec,
                  node_spec,
                  pl.BlockSpec((128, 128), lambda i: (0, 0)),
                  pl.BlockSpec((128, 1024), lambda i: (0, 0)),
                  pl.BlockSpec((1, 1024), lambda i: (0, 0)),
                  pl.BlockSpec((1024, 128), lambda i: (0, 0)),
                  pl.BlockSpec((1, 1), lambda i: (0, 0))],
        out_specs=node_spec,
        out_shape=jax.ShapeDtypeStruct((NPR, 128), jnp.float32),
    )(x, partials, partials, cntp, cntp, vpad, sum8, vw1k, vb1k, vw2k, vb2)


def kernel(pos, edge_index, edge_attr, v, params):
    row = edge_index[0]
    col = edge_index[1]
    rowp = jnp.concatenate(
        [row, jnp.full((EP - E,), NP - 1, jnp.int32)]).reshape(NW, CT, CHUNK)
    colp = jnp.concatenate(
        [col, jnp.zeros((EP - E,), jnp.int32)]).reshape(NW, CT, CHUNK)
    x8, v8 = _tc_prep(jnp.pad(pos.T, ((0, 0), (0, NP - N))),
                      jnp.pad(v.T, ((0, 0), (0, NP - N))))
    x = jnp.reshape(x8, (NPR, 128))
    vpad = jnp.reshape(v8, (NPR, 128))
    attr_i = jnp.reshape(
        jnp.pad(edge_attr, ((0, EP - E), (1, W - 5))).astype(jnp.bfloat16),
        (EPR, 128))
    zeros_nb = jnp.zeros((NB, W), jnp.float32)
    ones_g = jnp.ones((GSZ, W), jnp.float32)
    eye16 = jnp.eye(16, dtype=jnp.float32)
    sum8 = jnp.kron(eye16, jnp.ones((W, W), jnp.float32))

    cntp8 = _sc_count(ones_g, rowp, zeros_nb)
    cntp = jnp.reshape(cntp8, (NC * NPR, 128))

    for p in params:
        w1k = jnp.kron(eye16, jnp.pad(p['phi_W1'], ((0, W - 5), (0, 0))))
        b1k = jnp.tile(p['phi_b1'], 16)[None, :]
        w2k = jnp.kron(eye16,
                       jnp.broadcast_to(p['phi_W2'], (64, W)))
        vw1k = jnp.kron(eye16, jnp.pad(p['vel_W1'], ((0, W - 1), (0, 0))))
        vb1k = jnp.tile(p['vel_b1'], 16)[None, :]
        vw2k = jnp.kron(eye16,
                        jnp.broadcast_to(p['vel_W2'], (64, W)))
        vb2 = p['vel_b2'][None, :]
        xr8, xc8 = _sc_gather(jnp.reshape(x, (NP, W)), rowp, colp)
        m = _tc_mlp(jnp.reshape(xr8, (EPR, 128)), jnp.reshape(xc8, (EPR, 128)),
                    attr_i, sum8, w1k, b1k, w2k)
        partials8 = _sc_scatter(jnp.reshape(m, (EP, W)), rowp, zeros_nb)
        partials = jnp.reshape(partials8, (NC * NPR, 128))
        x = _tc_node(x, partials, cntp, vpad, sum8, vw1k, vb1k, vw2k, vb2)

    return jnp.reshape(x, (NP, W))[:N, :3]
